# Initial kernel scaffold; baseline (speedup 1.0000x reference)
import jax
import jax.numpy as jnp
from jax.experimental import pallas as pl


def kernel(z, W):
    raise NotImplementedError("write your pallas kernel here")


# speedup vs baseline: 2.1478x; 2.1478x over previous
import jax
import jax.numpy as jnp
from jax import lax
from jax.experimental import pallas as pl
from jax.experimental.pallas import tpu as pltpu
from jax.experimental.pallas import tpu_sc as plsc

K = 512
D = 64
B = 4096
ROWS = 1024
NB = B // ROWS
NC, NS = 2, 16
NW = NC * NS
BPW = B // NW
TAU = 1e-4
CAP = 160
DPAD = 128


def _tc_body(z_ref, w_ref, idx_ref, amb_ref, vq_ref, perp_ref, cnt_ref,
             mse_ref):
    b = pl.program_id(0)

    @pl.when(b == 0)
    def _init():
        cnt_ref[...] = jnp.zeros_like(cnt_ref)
        mse_ref[0, 0] = 0.0

    zb = z_ref[...]
    w = w_ref[...]
    zw = lax.dot_general(zb, w, (((1,), (1,)), ((), ())),
                         precision=lax.Precision.HIGHEST,
                         preferred_element_type=jnp.float32)
    wsq = lax.dot_general(jnp.ones((1, D), jnp.float32), w * w,
                          (((1,), (1,)), ((), ())),
                          precision=lax.Precision.HIGHEST,
                          preferred_element_type=jnp.float32)
    s = wsq - 2.0 * zw
    mn = jnp.min(s, axis=1, keepdims=True)
    it = lax.broadcasted_iota(jnp.int32, s.shape, 1)
    hit = s == mn
    idxc = jnp.min(jnp.where(hit, it, K), axis=1, keepdims=True)
    idx_ref[0] = idxc
    m2 = jnp.min(jnp.where(it == idxc, jnp.inf, s), axis=1, keepdims=True)
    amb_ref[0] = (m2 - mn <= TAU).astype(jnp.int32)

    zsq = jnp.sum(zb * zb, axis=1, keepdims=True)
    mse_ref[0, 0] += jnp.sum(zsq) + jnp.sum(mn)
    onehot = (it == idxc).astype(jnp.float32)
    cnt_ref[...] += jnp.sum(onehot, axis=0, keepdims=True)

    @pl.when(b == NB - 1)
    def _fini():
        avg = cnt_ref[...] * (1.0 / B)
        ent = jnp.sum(avg * jnp.log(avg + 1e-10))
        perp = jnp.exp(-ent)
        ratio = jnp.clip((K * 0.1) / (perp + 1e-10), 0.0, 10.0)
        mse = mse_ref[0, 0] * (1.0 / (B * D))
        vq_ref[0, 0] = 1.25 * mse + 0.1 * ratio
        perp_ref[0, 0] = perp


def _tc_vq(z, W):
    return pl.pallas_call(
        _tc_body,
        grid=(NB,),
        in_specs=[
            pl.BlockSpec((ROWS, D), lambda b: (b, 0)),
            pl.BlockSpec((K, D), lambda b: (0, 0)),
        ],
        out_specs=[
            pl.BlockSpec((1, ROWS, 1), lambda b: (b, 0, 0)),
            pl.BlockSpec((1, ROWS, 1), lambda b: (b, 0, 0)),
            pl.BlockSpec(memory_space=pltpu.SMEM),
            pl.BlockSpec(memory_space=pltpu.SMEM),
        ],
        out_shape=[
            jax.ShapeDtypeStruct((NB, ROWS, 1), jnp.int32),
            jax.ShapeDtypeStruct((NB, ROWS, 1), jnp.int32),
            jax.ShapeDtypeStruct((1, 1), jnp.float32),
            jax.ShapeDtypeStruct((1, 1), jnp.float32),
        ],
        scratch_shapes=[
            pltpu.VMEM((1, K), jnp.float32),
            pltpu.SMEM((1, 1), jnp.float32),
        ],
        compiler_params=pltpu.CompilerParams(
            dimension_semantics=("arbitrary",)),
    )(z, W)


def _recheck_body(zt_ref, wt_ref, out_ref):
    wt = wt_ref[...]
    it = lax.broadcasted_iota(jnp.int32, (1, K), 1)
    for i in range(CAP):
        zc = zt_ref[:, i:i + 1]
        df = zc - wt
        t = df * df
        acc = None
        for tt in range(8):
            q = t[8 * tt:8 * tt + 4, :] + t[8 * tt + 4:8 * tt + 8, :]
            r = q[0:2, :] + q[2:4, :]
            p = r[0:1, :] + r[1:2, :]
            acc = p if acc is None else acc + p
        mnv = jnp.min(acc, axis=1, keepdims=True)
        out_ref[0, i] = jnp.min(jnp.where(acc == mnv, it, K))


def _recheck(zt_amb, wt):
    return pl.pallas_call(
        _recheck_body,
        in_specs=[
            pl.BlockSpec((D, CAP), lambda: (0, 0)),
            pl.BlockSpec((D, K), lambda: (0, 0)),
        ],
        out_specs=pl.BlockSpec(memory_space=pltpu.SMEM),
        out_shape=jax.ShapeDtypeStruct((1, CAP), jnp.int32),
    )(zt_amb, wt)


def _sc_gather_body(w_hbm, idx_hbm, out_hbm, idx_v, rows_v, sem):
    wid = lax.axis_index("s") * NC + lax.axis_index("c")
    base = wid * BPW
    pltpu.sync_copy(idx_hbm.at[pl.ds(base, BPW)], idx_v)
    pltpu.async_copy(w_hbm.at[idx_v], rows_v, sem).wait()
    pltpu.sync_copy(rows_v, out_hbm.at[pl.ds(base, BPW)])


def _sc_gather(W_pad, idx):
    mesh = plsc.VectorSubcoreMesh(core_axis_name="c", subcore_axis_name="s")
    fn = pl.kernel(
        _sc_gather_body,
        mesh=mesh,
        out_type=jax.ShapeDtypeStruct((B, DPAD), jnp.float32),
        scratch_types=[
            pltpu.VMEM((BPW,), jnp.int32),
            pltpu.VMEM((BPW, DPAD), jnp.float32),
            pltpu.SemaphoreType.DMA,
        ],
    )
    return fn(W_pad, idx)


def kernel(z, W):
    idx3, amb3, vq, perp = _tc_vq(z, W)
    idx0 = idx3.reshape(B)
    amb = amb3.reshape(B)
    ids = jnp.nonzero(amb, size=CAP, fill_value=0)[0].astype(jnp.int32)
    zt_amb = z[ids].T
    idx_fix = _recheck(zt_amb, W.T)[0]
    idx = idx0.at[ids].set(idx_fix)
    W_pad = jnp.pad(W, ((0, 0), (0, DPAD - D)))
    z_q = _sc_gather(W_pad, idx)[:, :D]
    return (z_q, vq[0, 0], perp[0, 0])


# speedup vs baseline: 2.3718x; 1.1043x over previous
import jax
import jax.numpy as jnp
from jax import lax
from jax.experimental import pallas as pl
from jax.experimental.pallas import tpu as pltpu
from jax.experimental.pallas import tpu_sc as plsc

K = 512
D = 64
B = 4096
ROWS = 1024
NB = B // ROWS
NC, NS = 2, 16
NW = NC * NS
BPW = B // NW
TAU = 1e-4
CAP = 128
DPAD = 128


def _tc_body(z_ref, w_ref, idx_ref, amb_ref, vq_ref, perp_ref, cnt_ref,
             mse_ref):
    b = pl.program_id(0)

    @pl.when(b == 0)
    def _init():
        cnt_ref[...] = jnp.zeros_like(cnt_ref)
        mse_ref[0, 0] = 0.0

    zb = z_ref[...]
    w = w_ref[...]
    zw = lax.dot_general(zb, w, (((1,), (1,)), ((), ())),
                         precision=lax.Precision.HIGHEST,
                         preferred_element_type=jnp.float32)
    wsq = lax.dot_general(jnp.ones((1, D), jnp.float32), w * w,
                          (((1,), (1,)), ((), ())),
                          precision=lax.Precision.HIGHEST,
                          preferred_element_type=jnp.float32)
    s = wsq - 2.0 * zw
    mn = jnp.min(s, axis=1, keepdims=True)
    it = lax.broadcasted_iota(jnp.int32, s.shape, 1)
    hit = s == mn
    idxc = jnp.min(jnp.where(hit, it, K), axis=1, keepdims=True)
    idx_ref[0] = idxc
    m2 = jnp.min(jnp.where(it == idxc, jnp.inf, s), axis=1, keepdims=True)
    amb_ref[0] = (m2 - mn <= TAU).astype(jnp.int32)

    zsq = jnp.sum(zb * zb, axis=1, keepdims=True)
    mse_ref[0, 0] += jnp.sum(zsq) + jnp.sum(mn)
    onehot = (it == idxc).astype(jnp.float32)
    cnt_ref[...] += jnp.sum(onehot, axis=0, keepdims=True)

    @pl.when(b == NB - 1)
    def _fini():
        avg = cnt_ref[...] * (1.0 / B)
        ent = jnp.sum(avg * jnp.log(avg + 1e-10))
        perp = jnp.exp(-ent)
        ratio = jnp.clip((K * 0.1) / (perp + 1e-10), 0.0, 10.0)
        mse = mse_ref[0, 0] * (1.0 / (B * D))
        vq_ref[0, 0] = 1.25 * mse + 0.1 * ratio
        perp_ref[0, 0] = perp


def _tc_vq(z, W):
    return pl.pallas_call(
        _tc_body,
        grid=(NB,),
        in_specs=[
            pl.BlockSpec((ROWS, D), lambda b: (b, 0)),
            pl.BlockSpec((K, D), lambda b: (0, 0)),
        ],
        out_specs=[
            pl.BlockSpec((1, ROWS, 1), lambda b: (b, 0, 0)),
            pl.BlockSpec((1, ROWS, 1), lambda b: (b, 0, 0)),
            pl.BlockSpec(memory_space=pltpu.SMEM),
            pl.BlockSpec(memory_space=pltpu.SMEM),
        ],
        out_shape=[
            jax.ShapeDtypeStruct((NB, ROWS, 1), jnp.int32),
            jax.ShapeDtypeStruct((NB, ROWS, 1), jnp.int32),
            jax.ShapeDtypeStruct((1, 1), jnp.float32),
            jax.ShapeDtypeStruct((1, 1), jnp.float32),
        ],
        scratch_shapes=[
            pltpu.VMEM((1, K), jnp.float32),
            pltpu.SMEM((1, 1), jnp.float32),
        ],
        compiler_params=pltpu.CompilerParams(
            dimension_semantics=("arbitrary",)),
    )(z, W)


def _recheck_body(zt_ref, wt_ref, out_ref):
    wt = wt_ref[...]
    it = lax.broadcasted_iota(jnp.int32, (1, K), 1)
    for i in range(CAP):
        zc = zt_ref[:, i:i + 1]
        df = zc - wt
        t = df * df
        acc = None
        for tt in range(8):
            q = t[8 * tt:8 * tt + 4, :] + t[8 * tt + 4:8 * tt + 8, :]
            r = q[0:2, :] + q[2:4, :]
            p = r[0:1, :] + r[1:2, :]
            acc = p if acc is None else acc + p
        mnv = jnp.min(acc, axis=1, keepdims=True)
        out_ref[0, i] = jnp.min(jnp.where(acc == mnv, it, K))


def _recheck(zt_amb, wt):
    return pl.pallas_call(
        _recheck_body,
        in_specs=[
            pl.BlockSpec((D, CAP), lambda: (0, 0)),
            pl.BlockSpec((D, K), lambda: (0, 0)),
        ],
        out_specs=pl.BlockSpec(memory_space=pltpu.SMEM),
        out_shape=jax.ShapeDtypeStruct((1, CAP), jnp.int32),
    )(zt_amb, wt)


def _sc_gather_body(w_hbm, idx_hbm, out_hbm, idx_v, rows_v, sem):
    wid = lax.axis_index("s") * NC + lax.axis_index("c")
    base = wid * BPW
    pltpu.sync_copy(idx_hbm.at[pl.ds(base, BPW)], idx_v)
    pltpu.async_copy(w_hbm.at[idx_v], rows_v, sem).wait()
    pltpu.sync_copy(rows_v, out_hbm.at[pl.ds(base, BPW)])


def _sc_gather(W_pad, idx):
    mesh = plsc.VectorSubcoreMesh(core_axis_name="c", subcore_axis_name="s")
    fn = pl.kernel(
        _sc_gather_body,
        mesh=mesh,
        out_type=jax.ShapeDtypeStruct((B, DPAD), jnp.float32),
        scratch_types=[
            pltpu.VMEM((BPW,), jnp.int32),
            pltpu.VMEM((BPW, DPAD), jnp.float32),
            pltpu.SemaphoreType.DMA,
        ],
    )
    return fn(W_pad, idx)


def kernel(z, W):
    idx3, amb3, vq, perp = _tc_vq(z, W)
    idx0 = idx3.reshape(B)
    amb = amb3.reshape(B)
    ids = jnp.nonzero(amb, size=CAP, fill_value=0)[0].astype(jnp.int32)
    zt_amb = z[ids].T
    idx_fix = _recheck(zt_amb, W.T)[0]
    idx = idx0.at[ids].set(idx_fix)
    W_pad = jnp.pad(W, ((0, 0), (0, DPAD - D)))
    z_q = _sc_gather(W_pad, idx)[:, :D]
    return (z_q, vq[0, 0], perp[0, 0])


# speedup vs baseline: 3.3527x; 1.4136x over previous
import jax
import jax.numpy as jnp
from jax import lax
from jax.experimental import pallas as pl
from jax.experimental.pallas import tpu as pltpu
from jax.experimental.pallas import tpu_sc as plsc

K = 512
D = 64
B = 4096
ROWS = 1024
NB = B // ROWS
NC, NS = 2, 16
NW = NC * NS
BPW = B // NW
TAU = 1e-4
CAP = 128
DPAD = 128


def _tc_body(z_ref, w_ref, idx_ref, amb_ref, vq_ref, perp_ref, cnt_ref,
             mse_ref):
    b = pl.program_id(0)

    @pl.when(b == 0)
    def _init():
        cnt_ref[...] = jnp.zeros_like(cnt_ref)
        mse_ref[0, 0] = 0.0

    zb = z_ref[...]
    w = w_ref[...]
    zw = lax.dot_general(zb, w, (((1,), (1,)), ((), ())),
                         precision=lax.Precision.HIGHEST,
                         preferred_element_type=jnp.float32)
    wsq = lax.dot_general(jnp.ones((1, D), jnp.float32), w * w,
                          (((1,), (1,)), ((), ())),
                          precision=lax.Precision.HIGHEST,
                          preferred_element_type=jnp.float32)
    s = wsq - 2.0 * zw
    mn = jnp.min(s, axis=1, keepdims=True)
    it = lax.broadcasted_iota(jnp.int32, s.shape, 1)
    hit = s == mn
    idxc = jnp.min(jnp.where(hit, it, K), axis=1, keepdims=True)
    idx_ref[0] = idxc
    m2 = jnp.min(jnp.where(it == idxc, jnp.inf, s), axis=1, keepdims=True)
    amb_ref[0] = (m2 - mn <= TAU).astype(jnp.int32)

    zsq = jnp.sum(zb * zb, axis=1, keepdims=True)
    mse_ref[0, 0] += jnp.sum(zsq) + jnp.sum(mn)
    onehot = (it == idxc).astype(jnp.float32)
    cnt_ref[...] += jnp.sum(onehot, axis=0, keepdims=True)

    @pl.when(b == NB - 1)
    def _fini():
        avg = cnt_ref[...] * (1.0 / B)
        ent = jnp.sum(avg * jnp.log(avg + 1e-10))
        perp = jnp.exp(-ent)
        ratio = jnp.clip((K * 0.1) / (perp + 1e-10), 0.0, 10.0)
        mse = mse_ref[0, 0] * (1.0 / (B * D))
        vq_ref[0, 0] = 1.25 * mse + 0.1 * ratio
        perp_ref[0, 0] = perp


def _tc_vq(z, W):
    return pl.pallas_call(
        _tc_body,
        grid=(NB,),
        in_specs=[
            pl.BlockSpec((ROWS, D), lambda b: (b, 0)),
            pl.BlockSpec((K, D), lambda b: (0, 0)),
        ],
        out_specs=[
            pl.BlockSpec((1, ROWS, 1), lambda b: (b, 0, 0)),
            pl.BlockSpec((1, ROWS, 1), lambda b: (b, 0, 0)),
            pl.BlockSpec(memory_space=pltpu.SMEM),
            pl.BlockSpec(memory_space=pltpu.SMEM),
        ],
        out_shape=[
            jax.ShapeDtypeStruct((NB, ROWS, 1), jnp.int32),
            jax.ShapeDtypeStruct((NB, ROWS, 1), jnp.int32),
            jax.ShapeDtypeStruct((1, 1), jnp.float32),
            jax.ShapeDtypeStruct((1, 1), jnp.float32),
        ],
        scratch_shapes=[
            pltpu.VMEM((1, K), jnp.float32),
            pltpu.SMEM((1, 1), jnp.float32),
        ],
        compiler_params=pltpu.CompilerParams(
            dimension_semantics=("arbitrary",)),
    )(z, W)


def _recheck_body(zt_ref, wt_ref, out_ref):
    wt = wt_ref[...]
    it = lax.broadcasted_iota(jnp.int32, (1, 8, K), 2)
    for g in range(CAP // 8):
        zc = zt_ref[:, 8 * g:8 * g + 8]
        df = zc[:, :, None] - wt[:, None, :]
        t = df * df
        acc = None
        for tt in range(8):
            q = t[8 * tt:8 * tt + 4] + t[8 * tt + 4:8 * tt + 8]
            r = q[0:2] + q[2:4]
            p = r[0:1] + r[1:2]
            acc = p if acc is None else acc + p
        mnv = jnp.min(acc, axis=2, keepdims=True)
        iv = jnp.min(jnp.where(acc == mnv, it, K), axis=2, keepdims=True)
        out_ref[pl.ds(8 * g, 8), :] = iv.reshape(8, 1)


def _recheck(zt_amb, wt):
    return pl.pallas_call(
        _recheck_body,
        in_specs=[
            pl.BlockSpec((D, CAP), lambda: (0, 0)),
            pl.BlockSpec((D, K), lambda: (0, 0)),
        ],
        out_specs=pl.BlockSpec((CAP, 1), lambda: (0, 0)),
        out_shape=jax.ShapeDtypeStruct((CAP, 1), jnp.int32),
    )(zt_amb, wt)


def _sc_gather_body(w_hbm, idx_hbm, out_hbm, idx_v, rows_v, sem):
    wid = lax.axis_index("s") * NC + lax.axis_index("c")
    base = wid * BPW
    pltpu.sync_copy(idx_hbm.at[pl.ds(base, BPW)], idx_v)
    pltpu.async_copy(w_hbm.at[idx_v], rows_v, sem).wait()
    pltpu.sync_copy(rows_v, out_hbm.at[pl.ds(base, BPW)])


def _sc_gather(W_pad, idx):
    mesh = plsc.VectorSubcoreMesh(core_axis_name="c", subcore_axis_name="s")
    fn = pl.kernel(
        _sc_gather_body,
        mesh=mesh,
        out_type=jax.ShapeDtypeStruct((B, DPAD), jnp.float32),
        scratch_types=[
            pltpu.VMEM((BPW,), jnp.int32),
            pltpu.VMEM((BPW, DPAD), jnp.float32),
            pltpu.SemaphoreType.DMA,
        ],
    )
    return fn(W_pad, idx)


def kernel(z, W):
    idx3, amb3, vq, perp = _tc_vq(z, W)
    idx0 = idx3.reshape(B)
    amb = amb3.reshape(B)
    ids = jnp.nonzero(amb, size=CAP, fill_value=0)[0].astype(jnp.int32)
    zt_amb = z[ids].T
    idx_fix = _recheck(zt_amb, W.T).reshape(CAP)
    idx = idx0.at[ids].set(idx_fix)
    W_pad = jnp.pad(W, ((0, 0), (0, DPAD - D)))
    z_q = _sc_gather(W_pad, idx)[:, :D]
    return (z_q, vq[0, 0], perp[0, 0])
